# Initial kernel scaffold; baseline (speedup 1.0000x reference)
import jax
import jax.numpy as jnp
from jax.experimental import pallas as pl


def kernel(z, embedding):
    raise NotImplementedError("write your pallas kernel here")


# speedup vs baseline: 1.3506x; 1.3506x over previous
import functools

import jax
import jax.numpy as jnp
from jax import lax
from jax.experimental import pallas as pl
from jax.experimental.pallas import tpu as pltpu
from jax.experimental.pallas import tpu_sc as plsc

NUM_EMB = 8192
EMB_DIM = 256
TOKENS = 8192
TBLK = 1024
KBLK = 1024
NT = TOKENS // TBLK
NK = NUM_EMB // KBLK

_INT_MAX = 2**31 - 1


def _vq_body(zt_ref, emb_ref, ids_ref, loss_ref, zl_ref, vmax_ref, vidx_ref,
             lsum_ref):
    tb = pl.program_id(0)
    kb = pl.program_id(1)

    @pl.when(kb == 0)
    def _():
        zt = zt_ref[...]
        norm = jnp.sqrt(jnp.sum(zt * zt, axis=1, keepdims=True))
        norm = jnp.maximum(norm, 1e-12)
        zl_ref[...] = zt / norm

    cosT = lax.dot_general(emb_ref[...], zl_ref[...],
                           (((1,), (1,)), ((), ())),
                           preferred_element_type=jnp.float32)
    lmax = jnp.max(cosT, axis=0, keepdims=True)
    rows = lax.broadcasted_iota(jnp.int32, cosT.shape, 0) + kb * KBLK
    lidx = jnp.min(jnp.where(cosT == lmax, rows, _INT_MAX),
                   axis=0, keepdims=True)

    @pl.when(kb == 0)
    def _():
        vmax_ref[...] = lmax
        vidx_ref[...] = lidx

    @pl.when(kb > 0)
    def _():
        better = lmax > vmax_ref[...]
        vidx_ref[...] = jnp.where(better, lidx, vidx_ref[...])
        vmax_ref[...] = jnp.where(better, lmax, vmax_ref[...])

    @pl.when(kb == NK - 1)
    def _():
        ids_ref[...] = vidx_ref[...].reshape(1, 1, TBLK)
        s = jnp.sum(1.0 - vmax_ref[...])

        @pl.when(tb == 0)
        def _():
            lsum_ref[0, 0] = s

        @pl.when(tb > 0)
        def _():
            lsum_ref[0, 0] = lsum_ref[0, 0] + s

        @pl.when(tb == NT - 1)
        def _():
            loss_ref[...] = jnp.full((1, 1), lsum_ref[0, 0] / TOKENS,
                                     jnp.float32)


def _vq_argmax(zt, embedding):
    return pl.pallas_call(
        _vq_body,
        grid=(NT, NK),
        in_specs=[
            pl.BlockSpec((TBLK, EMB_DIM), lambda tb, kb: (tb, 0)),
            pl.BlockSpec((KBLK, EMB_DIM), lambda tb, kb: (kb, 0)),
        ],
        out_specs=[
            pl.BlockSpec((1, 1, TBLK), lambda tb, kb: (tb, 0, 0)),
            pl.BlockSpec((1, 1), lambda tb, kb: (0, 0)),
        ],
        out_shape=[
            jax.ShapeDtypeStruct((NT, 1, TBLK), jnp.int32),
            jax.ShapeDtypeStruct((1, 1), jnp.float32),
        ],
        scratch_shapes=[
            pltpu.VMEM((TBLK, EMB_DIM), jnp.float32),
            pltpu.VMEM((1, TBLK), jnp.float32),
            pltpu.VMEM((1, TBLK), jnp.int32),
            pltpu.SMEM((1, 1), jnp.float32),
        ],
    )(zt, embedding)


@functools.lru_cache(maxsize=1)
def _sc_gather():
    NC, NS = 2, 16
    NW = NC * NS
    b_per_w = TOKENS // NW
    mesh = plsc.VectorSubcoreMesh(core_axis_name="c", subcore_axis_name="s")

    @functools.partial(
        pl.kernel, mesh=mesh,
        out_type=jax.ShapeDtypeStruct((TOKENS, EMB_DIM), jnp.float32),
        scratch_types=[
            pltpu.VMEM((b_per_w,), jnp.int32),
            pltpu.VMEM((b_per_w, EMB_DIM), jnp.float32),
            pltpu.SemaphoreType.DMA,
        ],
    )
    def gather_rows(table_hbm, idx_hbm, out_hbm, idx_v, rows_v, sem):
        wid = lax.axis_index("s") * NC + lax.axis_index("c")
        base = wid * b_per_w
        pltpu.sync_copy(idx_hbm.at[pl.ds(base, b_per_w)], idx_v)
        pltpu.async_copy(table_hbm.at[idx_v], rows_v, sem).wait()
        pltpu.sync_copy(rows_v, out_hbm.at[pl.ds(base, b_per_w)])

    return gather_rows


def kernel(z, embedding):
    B, C, H, W = z.shape
    zt = z.reshape(B, C, H * W).transpose(0, 2, 1).reshape(B * H * W, C)
    ids3, loss11 = _vq_argmax(zt, embedding)
    flat_ids = ids3.reshape(-1)
    zq = _sc_gather()(embedding, flat_ids)
    z_q_out = zq.reshape(B, H * W, C).transpose(0, 2, 1).reshape(B, C, H, W)
    embed_ids = flat_ids.reshape(B, H, W)
    loss = loss11[0, 0]
    return (z_q_out, embed_ids, loss)


# speedup vs baseline: 1.9444x; 1.4397x over previous
import functools

import jax
import jax.numpy as jnp
from jax import lax
from jax.experimental import pallas as pl
from jax.experimental.pallas import tpu as pltpu
from jax.experimental.pallas import tpu_sc as plsc

NUM_EMB = 8192
EMB_DIM = 256
TOKENS = 8192
TBLK = 1024
KBLK = 1024
NT = TOKENS // TBLK
NK = NUM_EMB // KBLK

_INT_MAX = 2**31 - 1


def _vq_body(zt_ref, emb_ref, ids_ref, loss_ref, zl_ref, vmax_ref, vidx_ref,
             lsum_ref):
    tb = pl.program_id(0)
    kb = pl.program_id(1)

    @pl.when(kb == 0)
    def _():
        zt = zt_ref[...]
        norm = jnp.sqrt(jnp.sum(zt * zt, axis=1, keepdims=True))
        norm = jnp.maximum(norm, 1e-12)
        zl_ref[...] = zt / norm

    cosT = lax.dot_general(emb_ref[...], zl_ref[...],
                           (((1,), (1,)), ((), ())),
                           preferred_element_type=jnp.float32)
    lmax = jnp.max(cosT, axis=0, keepdims=True)
    lidx = jnp.argmax(cosT, axis=0).reshape(1, TBLK).astype(jnp.int32) + kb * KBLK

    @pl.when(kb == 0)
    def _():
        vmax_ref[...] = lmax
        vidx_ref[...] = lidx

    @pl.when(kb > 0)
    def _():
        better = lmax > vmax_ref[...]
        vidx_ref[...] = jnp.where(better, lidx, vidx_ref[...])
        vmax_ref[...] = jnp.where(better, lmax, vmax_ref[...])

    @pl.when(kb == NK - 1)
    def _():
        ids_ref[...] = vidx_ref[...].reshape(1, 1, TBLK)
        s = jnp.sum(1.0 - vmax_ref[...])

        @pl.when(tb == 0)
        def _():
            lsum_ref[0, 0] = s

        @pl.when(tb > 0)
        def _():
            lsum_ref[0, 0] = lsum_ref[0, 0] + s

        @pl.when(tb == NT - 1)
        def _():
            loss_ref[...] = jnp.full((1, 1), lsum_ref[0, 0] / TOKENS,
                                     jnp.float32)


def _vq_argmax(zt, embedding):
    return pl.pallas_call(
        _vq_body,
        grid=(NT, NK),
        in_specs=[
            pl.BlockSpec((TBLK, EMB_DIM), lambda tb, kb: (tb, 0)),
            pl.BlockSpec((KBLK, EMB_DIM), lambda tb, kb: (kb, 0)),
        ],
        out_specs=[
            pl.BlockSpec((1, 1, TBLK), lambda tb, kb: (tb, 0, 0)),
            pl.BlockSpec((1, 1), lambda tb, kb: (0, 0)),
        ],
        out_shape=[
            jax.ShapeDtypeStruct((NT, 1, TBLK), jnp.int32),
            jax.ShapeDtypeStruct((1, 1), jnp.float32),
        ],
        scratch_shapes=[
            pltpu.VMEM((TBLK, EMB_DIM), jnp.float32),
            pltpu.VMEM((1, TBLK), jnp.float32),
            pltpu.VMEM((1, TBLK), jnp.int32),
            pltpu.SMEM((1, 1), jnp.float32),
        ],
    )(zt, embedding)


@functools.lru_cache(maxsize=1)
def _sc_gather():
    NC, NS = 2, 16
    NW = NC * NS
    b_per_w = TOKENS // NW
    mesh = plsc.VectorSubcoreMesh(core_axis_name="c", subcore_axis_name="s")

    @functools.partial(
        pl.kernel, mesh=mesh,
        out_type=jax.ShapeDtypeStruct((TOKENS, EMB_DIM), jnp.float32),
        scratch_types=[
            pltpu.VMEM((b_per_w,), jnp.int32),
            pltpu.VMEM((b_per_w, EMB_DIM), jnp.float32),
            pltpu.SemaphoreType.DMA,
        ],
    )
    def gather_rows(table_hbm, idx_hbm, out_hbm, idx_v, rows_v, sem):
        wid = lax.axis_index("s") * NC + lax.axis_index("c")
        base = wid * b_per_w
        pltpu.sync_copy(idx_hbm.at[pl.ds(base, b_per_w)], idx_v)
        pltpu.async_copy(table_hbm.at[idx_v], rows_v, sem).wait()
        pltpu.sync_copy(rows_v, out_hbm.at[pl.ds(base, b_per_w)])

    return gather_rows


def kernel(z, embedding):
    B, C, H, W = z.shape
    zt = z.reshape(B, C, H * W).transpose(0, 2, 1).reshape(B * H * W, C)
    ids3, loss11 = _vq_argmax(zt, embedding)
    flat_ids = ids3.reshape(-1)
    zq = _sc_gather()(embedding, flat_ids)
    z_q_out = zq.reshape(B, H * W, C).transpose(0, 2, 1).reshape(B, C, H, W)
    embed_ids = flat_ids.reshape(B, H, W)
    loss = loss11[0, 0]
    return (z_q_out, embed_ids, loss)


# speedup vs baseline: 2.6459x; 1.3607x over previous
import functools

import jax
import jax.numpy as jnp
from jax import lax
from jax.experimental import pallas as pl
from jax.experimental.pallas import tpu as pltpu
from jax.experimental.pallas import tpu_sc as plsc

NUM_EMB = 8192
EMB_DIM = 256
TOKENS = 8192
TBLK = 1024
KBLK = 8192
NT = TOKENS // TBLK
NK = NUM_EMB // KBLK

_INT_MAX = 2**31 - 1


def _vq_body(zt_ref, emb_ref, ids_ref, loss_ref, zl_ref, vmax_ref, vidx_ref,
             lsum_ref):
    tb = pl.program_id(0)
    kb = pl.program_id(1)

    @pl.when(kb == 0)
    def _():
        zt = zt_ref[...]
        norm = jnp.sqrt(jnp.sum(zt * zt, axis=1, keepdims=True))
        norm = jnp.maximum(norm, 1e-12)
        zl_ref[...] = zt / norm

    cosT = lax.dot_general(emb_ref[...], zl_ref[...],
                           (((1,), (1,)), ((), ())),
                           preferred_element_type=jnp.float32)
    lmax = jnp.max(cosT, axis=0, keepdims=True)
    lidx = jnp.argmax(cosT, axis=0).reshape(1, TBLK).astype(jnp.int32) + kb * KBLK

    @pl.when(kb == 0)
    def _():
        vmax_ref[...] = lmax
        vidx_ref[...] = lidx

    @pl.when(kb > 0)
    def _():
        better = lmax > vmax_ref[...]
        vidx_ref[...] = jnp.where(better, lidx, vidx_ref[...])
        vmax_ref[...] = jnp.where(better, lmax, vmax_ref[...])

    @pl.when(kb == NK - 1)
    def _():
        ids_ref[...] = vidx_ref[...].reshape(1, 1, TBLK)
        s = jnp.sum(1.0 - vmax_ref[...])

        @pl.when(tb == 0)
        def _():
            lsum_ref[0, 0] = s

        @pl.when(tb > 0)
        def _():
            lsum_ref[0, 0] = lsum_ref[0, 0] + s

        @pl.when(tb == NT - 1)
        def _():
            loss_ref[...] = jnp.full((1, 1), lsum_ref[0, 0] / TOKENS,
                                     jnp.float32)


def _vq_argmax(zt, embedding):
    return pl.pallas_call(
        _vq_body,
        grid=(NT, NK),
        in_specs=[
            pl.BlockSpec((TBLK, EMB_DIM), lambda tb, kb: (tb, 0)),
            pl.BlockSpec((KBLK, EMB_DIM), lambda tb, kb: (kb, 0)),
        ],
        out_specs=[
            pl.BlockSpec((1, 1, TBLK), lambda tb, kb: (tb, 0, 0)),
            pl.BlockSpec((1, 1), lambda tb, kb: (0, 0)),
        ],
        out_shape=[
            jax.ShapeDtypeStruct((NT, 1, TBLK), jnp.int32),
            jax.ShapeDtypeStruct((1, 1), jnp.float32),
        ],
        scratch_shapes=[
            pltpu.VMEM((TBLK, EMB_DIM), jnp.float32),
            pltpu.VMEM((1, TBLK), jnp.float32),
            pltpu.VMEM((1, TBLK), jnp.int32),
            pltpu.SMEM((1, 1), jnp.float32),
        ],
    )(zt, embedding)


@functools.lru_cache(maxsize=1)
def _sc_gather():
    NC, NS = 2, 16
    NW = NC * NS
    b_per_w = TOKENS // NW
    mesh = plsc.VectorSubcoreMesh(core_axis_name="c", subcore_axis_name="s")

    @functools.partial(
        pl.kernel, mesh=mesh,
        out_type=jax.ShapeDtypeStruct((TOKENS, EMB_DIM), jnp.float32),
        scratch_types=[
            pltpu.VMEM((b_per_w,), jnp.int32),
            pltpu.VMEM((b_per_w, EMB_DIM), jnp.float32),
            pltpu.SemaphoreType.DMA,
        ],
    )
    def gather_rows(table_hbm, idx_hbm, out_hbm, idx_v, rows_v, sem):
        wid = lax.axis_index("s") * NC + lax.axis_index("c")
        base = wid * b_per_w
        pltpu.sync_copy(idx_hbm.at[pl.ds(base, b_per_w)], idx_v)
        pltpu.async_copy(table_hbm.at[idx_v], rows_v, sem).wait()
        pltpu.sync_copy(rows_v, out_hbm.at[pl.ds(base, b_per_w)])

    return gather_rows


def kernel(z, embedding):
    B, C, H, W = z.shape
    zt = z.reshape(B, C, H * W).transpose(0, 2, 1).reshape(B * H * W, C)
    ids3, loss11 = _vq_argmax(zt, embedding)
    flat_ids = ids3.reshape(-1)
    zq = _sc_gather()(embedding, flat_ids)
    z_q_out = zq.reshape(B, H * W, C).transpose(0, 2, 1).reshape(B, C, H, W)
    embed_ids = flat_ids.reshape(B, H, W)
    loss = loss11[0, 0]
    return (z_q_out, embed_ids, loss)
